# baseline (speedup 1.0000x reference)
import jax
import jax.numpy as jnp
from jax.experimental import pallas as pl
from jax.experimental.pallas import tpu as pltpu

B, T, NMEL = 4, 600, 80
STRIDE = 4
IN_DIM = NMEL * STRIDE
D_MODEL = 512
NCB = 4
EMB_DIM = 16
NUM_EMB = 8192
N = T // STRIDE
ROWS = B * N
EC = 2048
NE = NUM_EMB // EC
NE2 = NE // 2
RB = 200
NRB = ROWS // RB
NEG = -1e30


def _ln(x, eps=1e-6):
    m = jnp.mean(x, axis=-1, keepdims=True)
    s = x - m
    v = jnp.mean(s * s, axis=-1, keepdims=True)
    return s / jnp.sqrt(v + eps)


def _prep_kernel(x_ref, proj_ref, w_in_ref, b_in_ref,
                 w_ff1a_ref, w_ff1b_ref, b_ff1_ref,
                 w_ff2a_ref, w_ff2b_ref, b_ff2_ref,
                 enc_ref, xsn_ref):
    x = x_ref[...]
    y = _ln(x)
    xs = jnp.dot(y, proj_ref[...], preferred_element_type=jnp.float32)
    nrm = jnp.sqrt(jnp.sum(xs * xs, axis=-1, keepdims=True))
    xsn_ref[...] = xs / (nrm + 1e-8)
    h1 = jnp.dot(x, w_in_ref[...], preferred_element_type=jnp.float32) \
        + b_in_ref[...]
    t = _ln(h1)
    FH = 2 * D_MODEL
    b_ff1 = b_ff1_ref[...]
    f0 = jax.nn.gelu(
        jnp.dot(t, w_ff1a_ref[...], preferred_element_type=jnp.float32)
        + b_ff1[:, :FH])
    f1 = jax.nn.gelu(
        jnp.dot(t, w_ff1b_ref[...], preferred_element_type=jnp.float32)
        + b_ff1[:, FH:])
    h2 = h1 \
        + jnp.dot(f0, w_ff2a_ref[...], preferred_element_type=jnp.float32) \
        + jnp.dot(f1, w_ff2b_ref[...], preferred_element_type=jnp.float32) \
        + b_ff2_ref[...]
    enc_ref[...] = _ln(h2)


def _codes_kernel(xsn_ref, embt_ref, tmask_ref, t0_ref,
                  tgt_ref, vals_ref, runmin_ref, runidx_ref):
    ec = pl.program_id(0)

    @pl.when(ec == 0)
    def _():
        runmin_ref[...] = jnp.full_like(runmin_ref, jnp.inf)
        runidx_ref[...] = jnp.full_like(runidx_ref, NUM_EMB)

    iota = jax.lax.broadcasted_iota(jnp.int32, (ROWS, EC), 1)
    for cb in range(NCB):
        xq = xsn_ref[cb]
        et = embt_ref[cb]
        c2 = jnp.sum(et * et, axis=0, keepdims=True)
        d = c2 - 2.0 * jnp.dot(xq, et, preferred_element_type=jnp.float32)
        m = jnp.min(d, axis=1, keepdims=True)
        idx = jnp.min(jnp.where(d == m, iota, NUM_EMB), axis=1,
                      keepdims=True) + ec * EC
        col = pl.ds(cb, 1)
        upd = m < runmin_ref[:, col]
        runidx_ref[:, col] = jnp.where(upd, idx, runidx_ref[:, col])
        runmin_ref[:, col] = jnp.where(upd, m, runmin_ref[:, col])

    @pl.when(ec == NE - 1)
    def _():
        t0 = t0_ref[...]
        tmask = tmask_ref[...]
        for cb in range(NCB):
            codes = runidx_ref[:, pl.ds(cb, 1)]
            tgt_ref[cb] = jnp.concatenate([codes[1:, :], codes[:1, :]],
                                          axis=0)
            vals_ref[cb] = jnp.where(t0 != 0, -1,
                                     jnp.where(tmask != 0, codes, 0))


def _loss_kernel(enc_ref, w_ref,
                 tgt_ref, vals_ref, valid_ref, msum_ref,
                 nll_ref, corr_ref, uniq_ref, pres_ref):
    cb = pl.program_id(0)
    r = pl.program_id(1)

    @pl.when(jnp.logical_and(cb == 0, r == 0))
    def _():
        nll_ref[...] = jnp.zeros_like(nll_ref)
        corr_ref[...] = jnp.zeros_like(corr_ref)
        uniq_ref[...] = jnp.zeros_like(uniq_ref)
        pres_ref[...] = jnp.zeros_like(pres_ref)

    enc = enc_ref[...]
    Ls = [jnp.dot(enc, w_ref[0, :, h * EC:(h + 1) * EC],
                  preferred_element_type=jnp.float32)
          for h in range(NE)]
    iota = jax.lax.broadcasted_iota(jnp.int32, (RB, EC), 1)
    tgt = tgt_ref[0]
    vals = vals_ref[0]
    v = valid_ref[...]

    m = jnp.max(Ls[0], axis=1, keepdims=True)
    for L in Ls[1:]:
        m = jnp.maximum(m, jnp.max(L, axis=1, keepdims=True))
    se = jnp.sum(jnp.exp(Ls[0] - m), axis=1, keepdims=True)
    for L in Ls[1:]:
        se = se + jnp.sum(jnp.exp(L - m), axis=1, keepdims=True)
    lse = m + jnp.log(se)

    tl = jnp.sum(jnp.where(iota == tgt, Ls[0], 0.0), axis=1, keepdims=True)
    for h, L in enumerate(Ls[1:], start=1):
        tl = tl + jnp.sum(jnp.where(iota == (tgt - h * EC), L, 0.0),
                          axis=1, keepdims=True)

    nll_ref[...] = nll_ref[...] + jnp.sum(v * (lse - tl))
    corr_ref[...] = corr_ref[...] + jnp.sum(
        v * (tl == m).astype(jnp.float32))

    for h, L in enumerate(Ls):
        pres_ref[h:h + 1, :] = pres_ref[h:h + 1, :] + jnp.sum(
            (iota == (vals - h * EC)).astype(jnp.float32),
            axis=0, keepdims=True)

    @pl.when(jnp.logical_and(cb == NCB - 1, r == NRB - 1))
    def _():
        uniq_ref[...] = jnp.zeros_like(uniq_ref) + jnp.sum(
            (pres_ref[...] > 0).astype(jnp.float32))
        denom = msum_ref[0, 0] * NCB
        nll_ref[...] = nll_ref[...] / denom
        corr_ref[...] = corr_ref[...] / denom


def kernel(feats, feats_lengths, projection, embeddings, W_in, b_in,
           W_ff1, b_ff1, W_ff2, b_ff2, top_n_out):
    x = feats.reshape(ROWS, IN_DIM)
    embT = jnp.transpose(embeddings, (1, 2, 0))

    FH = 2 * D_MODEL
    enc, xsn = pl.pallas_call(
        _prep_kernel,
        grid=(1,),
        in_specs=[
            pl.BlockSpec((ROWS, IN_DIM), lambda i: (0, 0)),
            pl.BlockSpec((IN_DIM, NCB * EMB_DIM), lambda i: (0, 0)),
            pl.BlockSpec((IN_DIM, D_MODEL), lambda i: (0, 0)),
            pl.BlockSpec((1, D_MODEL), lambda i: (0, 0)),
            pl.BlockSpec((D_MODEL, FH), lambda i: (0, 0)),
            pl.BlockSpec((D_MODEL, FH), lambda i: (0, 1)),
            pl.BlockSpec((1, 4 * D_MODEL), lambda i: (0, 0)),
            pl.BlockSpec((FH, D_MODEL), lambda i: (0, 0)),
            pl.BlockSpec((FH, D_MODEL), lambda i: (1, 0)),
            pl.BlockSpec((1, D_MODEL), lambda i: (0, 0)),
        ],
        out_specs=[
            pl.BlockSpec((ROWS, D_MODEL), lambda i: (0, 0)),
            pl.BlockSpec((ROWS, NCB * EMB_DIM), lambda i: (0, 0)),
        ],
        out_shape=[jax.ShapeDtypeStruct((ROWS, D_MODEL), jnp.float32),
                   jax.ShapeDtypeStruct((ROWS, NCB * EMB_DIM), jnp.float32)],
    )(x, projection, W_in, b_in.reshape(1, -1),
      W_ff1, W_ff1, b_ff1.reshape(1, -1), W_ff2, W_ff2,
      b_ff2.reshape(1, -1))

    xsn3 = jnp.transpose(xsn.reshape(ROWS, NCB, EMB_DIM), (1, 0, 2))

    lim = feats_lengths // STRIDE
    t_idx = jnp.arange(N)
    validf = ((t_idx[None, :] + 1) < lim[:, None]).astype(jnp.float32) \
        .reshape(ROWS, 1)
    tmask_i = (t_idx[None, :] < lim[:, None]).astype(jnp.int32) \
        .reshape(ROWS, 1)
    t0_i = (jnp.arange(ROWS) % N == 0).astype(jnp.int32).reshape(ROWS, 1)
    mask_sum = jnp.sum(jnp.maximum(lim - 1, 0).astype(jnp.float32))
    msum = mask_sum.reshape(1, 1)

    tgt, vals2 = pl.pallas_call(
        _codes_kernel,
        grid=(NE,),
        in_specs=[
            pl.BlockSpec((NCB, ROWS, EMB_DIM), lambda ec: (0, 0, 0)),
            pl.BlockSpec((NCB, EMB_DIM, EC), lambda ec: (0, 0, ec)),
            pl.BlockSpec((ROWS, 1), lambda ec: (0, 0)),
            pl.BlockSpec((ROWS, 1), lambda ec: (0, 0)),
        ],
        out_specs=[pl.BlockSpec((NCB, ROWS, 1), lambda ec: (0, 0, 0))] * 2,
        out_shape=[jax.ShapeDtypeStruct((NCB, ROWS, 1), jnp.int32)] * 2,
        scratch_shapes=[pltpu.VMEM((ROWS, NCB), jnp.float32),
                        pltpu.VMEM((ROWS, NCB), jnp.int32)],
    )(xsn3, embT, tmask_i, t0_i)

    W = top_n_out[0]

    lossv, acc, uniq = pl.pallas_call(
        _loss_kernel,
        grid=(NCB, NRB),
        in_specs=[
            pl.BlockSpec((RB, D_MODEL), lambda cb, r: (r, 0)),
            pl.BlockSpec((1, D_MODEL, NUM_EMB), lambda cb, r: (cb, 0, 0)),
            pl.BlockSpec((1, RB, 1), lambda cb, r: (cb, r, 0)),
            pl.BlockSpec((1, RB, 1), lambda cb, r: (cb, r, 0)),
            pl.BlockSpec((RB, 1), lambda cb, r: (r, 0)),
            pl.BlockSpec((1, 1), lambda cb, r: (0, 0)),
        ],
        out_specs=[pl.BlockSpec((1, 1), lambda cb, r: (0, 0))] * 3,
        out_shape=[jax.ShapeDtypeStruct((1, 1), jnp.float32)] * 3,
        scratch_shapes=[pltpu.VMEM((NE, EC), jnp.float32)],
    )(enc, W, tgt, vals2, validf, msum)

    num_codes = mask_sum * NCB
    return (acc[0, 0], lossv[0, 0], num_codes, uniq[0, 0].astype(jnp.int32))
